# Initial kernel scaffold; baseline (speedup 1.0000x reference)
import jax
import jax.numpy as jnp
from jax.experimental import pallas as pl


def kernel(user_ids, book_ids, user_table, book_table, W0, b0, W1, b1, W2, b2, W3, b3):
    raise NotImplementedError("write your pallas kernel here")


# speedup vs baseline: 2.3302x; 2.3302x over previous
import functools

import jax
import jax.numpy as jnp
from jax import lax
from jax.experimental import pallas as pl
from jax.experimental.pallas import tpu as pltpu
from jax.experimental.pallas import tpu_sc as plsc

B = 16384
D = 128
NC = 2
NS = 16
NW = NC * NS
BPW = B // NW
CH = 128
NCH = BPW // CH


@functools.cache
def _make_gather():
    mesh = plsc.VectorSubcoreMesh(core_axis_name="c", subcore_axis_name="s")

    @functools.partial(
        pl.kernel,
        mesh=mesh,
        out_type=[
            jax.ShapeDtypeStruct((B, D), jnp.float32),
            jax.ShapeDtypeStruct((B, D), jnp.float32),
        ],
        scratch_types=[
            pltpu.VMEM((CH,), jnp.int32),
            pltpu.VMEM((CH,), jnp.int32),
            pltpu.VMEM((CH, D), jnp.float32),
            pltpu.VMEM((CH, D), jnp.float32),
            pltpu.SemaphoreType.DMA,
            pltpu.SemaphoreType.DMA,
        ],
    )
    def gather_k(uids, bids, utab, btab, u_out, b_out,
                 uidx_v, bidx_v, urows_v, brows_v, sem_u, sem_b):
        wid = lax.axis_index("s") * NC + lax.axis_index("c")
        base = wid * BPW
        for c in range(NCH):
            off = base + c * CH
            pltpu.sync_copy(uids.at[pl.ds(off, CH)], uidx_v)
            pltpu.sync_copy(bids.at[pl.ds(off, CH)], bidx_v)
            cu = pltpu.async_copy(utab.at[uidx_v], urows_v, sem_u)
            cb = pltpu.async_copy(btab.at[bidx_v], brows_v, sem_b)
            cu.wait()
            cb.wait()
            pltpu.sync_copy(urows_v, u_out.at[pl.ds(off, CH)])
            pltpu.sync_copy(brows_v, b_out.at[pl.ds(off, CH)])

    return gather_k


CHUNK = 1024
GRID = B // CHUNK


def _mlp_body(u_ref, b_ref, w0a_ref, w0b_ref, b0_ref, w1_ref, b1_ref,
              w2_ref, b2_ref, w3_ref, b3_ref, out_ref):
    h = jnp.dot(u_ref[...], w0a_ref[...], preferred_element_type=jnp.float32)
    h = h + jnp.dot(b_ref[...], w0b_ref[...], preferred_element_type=jnp.float32)
    h = jnp.maximum(h + b0_ref[...][None, :], 0.0)
    h = jnp.dot(h, w1_ref[...], preferred_element_type=jnp.float32)
    h = jnp.maximum(h + b1_ref[...][None, :], 0.0)
    h = jnp.dot(h, w2_ref[...], preferred_element_type=jnp.float32)
    h = jnp.maximum(h + b2_ref[...][None, :], 0.0)
    out_ref[...] = jnp.dot(h, w3_ref[...], preferred_element_type=jnp.float32) + b3_ref[0, 0]


def _mlp(u, bk, W0a, W0b, b0, W1, b1, W2, b2, w3, b3):
    return pl.pallas_call(
        _mlp_body,
        grid=(GRID,),
        in_specs=[
            pl.BlockSpec((CHUNK, D), lambda i: (i, 0)),
            pl.BlockSpec((CHUNK, D), lambda i: (i, 0)),
            pl.BlockSpec((D, 512), lambda i: (0, 0)),
            pl.BlockSpec((D, 512), lambda i: (0, 0)),
            pl.BlockSpec((512,), lambda i: (0,)),
            pl.BlockSpec((512, 256), lambda i: (0, 0)),
            pl.BlockSpec((256,), lambda i: (0,)),
            pl.BlockSpec((256, 128), lambda i: (0, 0)),
            pl.BlockSpec((128,), lambda i: (0,)),
            pl.BlockSpec((128, 1), lambda i: (0, 0)),
            pl.BlockSpec((1, 1), lambda i: (0, 0)),
        ],
        out_specs=pl.BlockSpec((CHUNK, 1), lambda i: (i, 0)),
        out_shape=jax.ShapeDtypeStruct((B, 1), jnp.float32),
        compiler_params=pltpu.CompilerParams(
            dimension_semantics=("parallel",),
        ),
    )(u, bk, W0a, W0b, b0, W1, b1, W2, b2, w3, b3)


def kernel(user_ids, book_ids, user_table, book_table,
           W0, b0, W1, b1, W2, b2, W3, b3):
    u, bk = _make_gather()(user_ids.astype(jnp.int32), book_ids.astype(jnp.int32),
                           user_table, book_table)
    out = _mlp(u, bk, W0[:D], W0[D:], b0, W1, b1, W2, b2, W3, b3[None, :])
    return out[:, 0]


# speedup vs baseline: 2.4774x; 1.0631x over previous
import functools

import jax
import jax.numpy as jnp
from jax import lax
from jax.experimental import pallas as pl
from jax.experimental.pallas import tpu as pltpu
from jax.experimental.pallas import tpu_sc as plsc

B = 16384
D = 128
NC = 2
NS = 16
NW = NC * NS
CH = 128


@functools.cache
def _make_gather(nb):
    bpw = nb // NW
    nch = bpw // CH
    mesh = plsc.VectorSubcoreMesh(core_axis_name="c", subcore_axis_name="s")

    @functools.partial(
        pl.kernel,
        mesh=mesh,
        out_type=[
            jax.ShapeDtypeStruct((nb, D), jnp.float32),
            jax.ShapeDtypeStruct((nb, D), jnp.float32),
        ],
        scratch_types=[
            pltpu.VMEM((CH,), jnp.int32),
            pltpu.VMEM((CH,), jnp.int32),
            pltpu.VMEM((CH, D), jnp.float32),
            pltpu.VMEM((CH, D), jnp.float32),
            pltpu.SemaphoreType.DMA,
            pltpu.SemaphoreType.DMA,
        ],
    )
    def gather_k(uids, bids, utab, btab, u_out, b_out,
                 uidx_v, bidx_v, urows_v, brows_v, sem_u, sem_b):
        wid = lax.axis_index("s") * NC + lax.axis_index("c")
        base = wid * bpw
        for c in range(nch):
            off = base + c * CH
            pltpu.sync_copy(uids.at[pl.ds(off, CH)], uidx_v)
            pltpu.sync_copy(bids.at[pl.ds(off, CH)], bidx_v)
            cu = pltpu.async_copy(utab.at[uidx_v], urows_v, sem_u)
            cb = pltpu.async_copy(btab.at[bidx_v], brows_v, sem_b)
            cu.wait()
            cb.wait()
            pltpu.sync_copy(urows_v, u_out.at[pl.ds(off, CH)])
            pltpu.sync_copy(brows_v, b_out.at[pl.ds(off, CH)])

    return gather_k


CHUNK = 1024


def _mlp_body(u_ref, b_ref, w0a_ref, w0b_ref, b0_ref, w1_ref, b1_ref,
              w2_ref, b2_ref, w3_ref, b3_ref, out_ref):
    bf = jnp.bfloat16
    h = jnp.dot(u_ref[...].astype(bf), w0a_ref[...],
                preferred_element_type=jnp.float32)
    h = h + jnp.dot(b_ref[...].astype(bf), w0b_ref[...],
                    preferred_element_type=jnp.float32)
    h = jnp.maximum(h + b0_ref[...][None, :], 0.0)
    h = jnp.dot(h.astype(bf), w1_ref[...], preferred_element_type=jnp.float32)
    h = jnp.maximum(h + b1_ref[...][None, :], 0.0)
    h = jnp.dot(h.astype(bf), w2_ref[...], preferred_element_type=jnp.float32)
    h = jnp.maximum(h + b2_ref[...][None, :], 0.0)
    out_ref[...] = jnp.dot(h.astype(bf), w3_ref[...],
                           preferred_element_type=jnp.float32) + b3_ref[0, 0]


def _mlp(nb, u, bk, W0a, W0b, b0, W1, b1, W2, b2, w3, b3):
    return pl.pallas_call(
        _mlp_body,
        grid=(nb // CHUNK,),
        in_specs=[
            pl.BlockSpec((CHUNK, D), lambda i: (i, 0)),
            pl.BlockSpec((CHUNK, D), lambda i: (i, 0)),
            pl.BlockSpec((D, 512), lambda i: (0, 0)),
            pl.BlockSpec((D, 512), lambda i: (0, 0)),
            pl.BlockSpec((512,), lambda i: (0,)),
            pl.BlockSpec((512, 256), lambda i: (0, 0)),
            pl.BlockSpec((256,), lambda i: (0,)),
            pl.BlockSpec((256, 128), lambda i: (0, 0)),
            pl.BlockSpec((128,), lambda i: (0,)),
            pl.BlockSpec((128, 1), lambda i: (0, 0)),
            pl.BlockSpec((1, 1), lambda i: (0, 0)),
        ],
        out_specs=pl.BlockSpec((CHUNK, 1), lambda i: (i, 0)),
        out_shape=jax.ShapeDtypeStruct((nb, 1), jnp.float32),
        compiler_params=pltpu.CompilerParams(
            dimension_semantics=("parallel",),
        ),
    )(u, bk, W0a, W0b, b0, W1, b1, W2, b2, w3, b3)


NSPLIT = 2


def kernel(user_ids, book_ids, user_table, book_table,
           W0, b0, W1, b1, W2, b2, W3, b3):
    bf = jnp.bfloat16
    w = (W0[:D].astype(bf), W0[D:].astype(bf), b0, W1.astype(bf), b1,
         W2.astype(bf), b2, W3.astype(bf), b3[None, :])
    h = B // NSPLIT
    uids = user_ids.astype(jnp.int32)
    bids = book_ids.astype(jnp.int32)
    outs = []
    for s in range(NSPLIT):
        u, bk = _make_gather(h)(uids[s * h:(s + 1) * h],
                                bids[s * h:(s + 1) * h],
                                user_table, book_table)
        outs.append(_mlp(h, u, bk, *w))
    return jnp.concatenate(outs, axis=0)[:, 0]


# speedup vs baseline: 2.6410x; 1.0661x over previous
import functools

import jax
import jax.numpy as jnp
from jax import lax
from jax.experimental import pallas as pl
from jax.experimental.pallas import tpu as pltpu
from jax.experimental.pallas import tpu_sc as plsc

B = 16384
D = 128
D2 = 2 * D
NC = 2
NS = 16
NW = NC * NS
CH = 128


@functools.cache
def _make_gather(nb):
    bpw = nb // NW
    nch = bpw // CH
    mesh = plsc.VectorSubcoreMesh(core_axis_name="c", subcore_axis_name="s")

    @functools.partial(
        pl.kernel,
        mesh=mesh,
        out_type=jax.ShapeDtypeStruct((nb, D2), jnp.float32),
        scratch_types=[
            pltpu.VMEM((nch, CH), jnp.int32),
            pltpu.VMEM((nch, CH), jnp.int32),
            pltpu.VMEM((CH, D), jnp.float32),
            pltpu.VMEM((CH, D), jnp.float32),
            pltpu.VMEM((CH, D), jnp.float32),
            pltpu.VMEM((CH, D), jnp.float32),
            pltpu.SemaphoreType.DMA,
            pltpu.SemaphoreType.DMA,
        ],
    )
    def gather_k(uids2, bids2, utab, btab, x_out,
                 uidx_v, bidx_v, ur0, ur1, br0, br1, sg, sw):
        wid = lax.axis_index("s") * NC + lax.axis_index("c")
        cbase = wid * nch
        pltpu.sync_copy(uids2.at[pl.ds(cbase, nch)], uidx_v)
        pltpu.sync_copy(bids2.at[pl.ds(cbase, nch)], bidx_v)
        ubufs, bbufs = (ur0, ur1), (br0, br1)
        gathers = {}
        writes = {}

        def fire_gather(c):
            gu = pltpu.async_copy(utab.at[uidx_v.at[c]], ubufs[c % 2], sg)
            gb = pltpu.async_copy(btab.at[bidx_v.at[c]], bbufs[c % 2], sg)
            gathers[c] = (gu, gb)

        fire_gather(0)
        for c in range(nch):
            if c + 1 < nch:
                if c - 1 >= 0:
                    for w in writes.pop(c - 1):
                        w.wait()
                fire_gather(c + 1)
            gu, gb = gathers.pop(c)
            gu.wait()
            gb.wait()
            off = (cbase + c) * CH
            wu = pltpu.async_copy(
                ubufs[c % 2], x_out.at[pl.ds(off, CH), pl.ds(0, D)], sw)
            wb = pltpu.async_copy(
                bbufs[c % 2], x_out.at[pl.ds(off, CH), pl.ds(D, D)], sw)
            writes[c] = (wu, wb)
        for c in sorted(writes):
            for w in writes[c]:
                w.wait()

    return gather_k


CHUNK = 1024


def _mlp_body(x_ref, w0_ref, b0_ref, w1_ref, b1_ref,
              w2_ref, b2_ref, w3_ref, b3_ref, out_ref):
    bf = jnp.bfloat16
    h = jnp.dot(x_ref[...].astype(bf), w0_ref[...],
                preferred_element_type=jnp.float32)
    h = jnp.maximum(h + b0_ref[...][None, :], 0.0)
    h = jnp.dot(h.astype(bf), w1_ref[...], preferred_element_type=jnp.float32)
    h = jnp.maximum(h + b1_ref[...][None, :], 0.0)
    h = jnp.dot(h.astype(bf), w2_ref[...], preferred_element_type=jnp.float32)
    h = jnp.maximum(h + b2_ref[...][None, :], 0.0)
    out_ref[...] = jnp.dot(h.astype(bf), w3_ref[...],
                           preferred_element_type=jnp.float32) + b3_ref[0, 0]


def _mlp(nb, x, W0, b0, W1, b1, W2, b2, w3, b3):
    return pl.pallas_call(
        _mlp_body,
        grid=(nb // CHUNK,),
        in_specs=[
            pl.BlockSpec((CHUNK, D2), lambda i: (i, 0)),
            pl.BlockSpec((D2, 512), lambda i: (0, 0)),
            pl.BlockSpec((512,), lambda i: (0,)),
            pl.BlockSpec((512, 256), lambda i: (0, 0)),
            pl.BlockSpec((256,), lambda i: (0,)),
            pl.BlockSpec((256, 128), lambda i: (0, 0)),
            pl.BlockSpec((128,), lambda i: (0,)),
            pl.BlockSpec((128, 1), lambda i: (0, 0)),
            pl.BlockSpec((1, 1), lambda i: (0, 0)),
        ],
        out_specs=pl.BlockSpec((CHUNK, 1), lambda i: (i, 0)),
        out_shape=jax.ShapeDtypeStruct((nb, 1), jnp.float32),
        compiler_params=pltpu.CompilerParams(
            dimension_semantics=("parallel",),
        ),
    )(x, W0, b0, W1, b1, W2, b2, w3, b3)


NSPLIT = 2


def kernel(user_ids, book_ids, user_table, book_table,
           W0, b0, W1, b1, W2, b2, W3, b3):
    bf = jnp.bfloat16
    w = (W0.astype(bf), b0, W1.astype(bf), b1, W2.astype(bf), b2,
         W3.astype(bf), b3[None, :])
    h = B // NSPLIT
    uids2 = user_ids.astype(jnp.int32).reshape(-1, CH)
    bids2 = book_ids.astype(jnp.int32).reshape(-1, CH)
    rph = h // CH
    outs = []
    for s in range(NSPLIT):
        x = _make_gather(h)(uids2[s * rph:(s + 1) * rph],
                            bids2[s * rph:(s + 1) * rph],
                            user_table, book_table)
        outs.append(_mlp(h, x, *w))
    return jnp.concatenate(outs, axis=0)[:, 0]


# speedup vs baseline: 2.7322x; 1.0345x over previous
import functools

import jax
import jax.numpy as jnp
from jax import lax
from jax.experimental import pallas as pl
from jax.experimental.pallas import tpu as pltpu
from jax.experimental.pallas import tpu_sc as plsc

B = 16384
D = 128
D2 = 2 * D
NC = 2
NS = 16
NW = NC * NS
CH = 128


@functools.cache
def _make_gather(nb, cbase0):
    bpw = nb // NW
    nch = bpw // CH
    mesh = plsc.VectorSubcoreMesh(core_axis_name="c", subcore_axis_name="s")

    @functools.partial(
        pl.kernel,
        mesh=mesh,
        out_type=jax.ShapeDtypeStruct((nb, D2), jnp.float32),
        scratch_types=[
            pltpu.VMEM((nch, CH), jnp.int32),
            pltpu.VMEM((nch, CH), jnp.int32),
            pltpu.VMEM((CH, D), jnp.float32),
            pltpu.VMEM((CH, D), jnp.float32),
            pltpu.VMEM((CH, D), jnp.float32),
            pltpu.VMEM((CH, D), jnp.float32),
            pltpu.SemaphoreType.DMA,
            pltpu.SemaphoreType.DMA,
        ],
    )
    def gather_k(uids2, bids2, utab, btab, x_out,
                 uidx_v, bidx_v, ur0, ur1, br0, br1, sg, sw):
        wid = lax.axis_index("s") * NC + lax.axis_index("c")
        cbase = wid * nch
        pltpu.sync_copy(uids2.at[pl.ds(cbase0 + cbase, nch)], uidx_v)
        pltpu.sync_copy(bids2.at[pl.ds(cbase0 + cbase, nch)], bidx_v)
        ubufs, bbufs = (ur0, ur1), (br0, br1)
        gathers = {}
        writes = {}

        def fire_gather(c):
            gu = pltpu.async_copy(utab.at[uidx_v.at[c]], ubufs[c % 2], sg)
            gb = pltpu.async_copy(btab.at[bidx_v.at[c]], bbufs[c % 2], sg)
            gathers[c] = (gu, gb)

        fire_gather(0)
        for c in range(nch):
            if c + 1 < nch:
                if c - 1 >= 0:
                    for w in writes.pop(c - 1):
                        w.wait()
                fire_gather(c + 1)
            gu, gb = gathers.pop(c)
            gu.wait()
            gb.wait()
            off = (cbase + c) * CH
            wu = pltpu.async_copy(
                ubufs[c % 2], x_out.at[pl.ds(off, CH), pl.ds(0, D)], sw)
            wb = pltpu.async_copy(
                bbufs[c % 2], x_out.at[pl.ds(off, CH), pl.ds(D, D)], sw)
            writes[c] = (wu, wb)
        for c in sorted(writes):
            for w in writes[c]:
                w.wait()

    return gather_k


CHUNK = 1024


def _mlp_body(x_ref, w0_ref, b0_ref, w1_ref, b1_ref,
              w2_ref, b2_ref, w3_ref, b3_ref, out_ref):
    bf = jnp.bfloat16
    h = jnp.dot(x_ref[...].astype(bf), w0_ref[...],
                preferred_element_type=jnp.float32)
    h = jnp.maximum(h + b0_ref[...][None, :], 0.0)
    h = jnp.dot(h.astype(bf), w1_ref[...], preferred_element_type=jnp.float32)
    h = jnp.maximum(h + b1_ref[...][None, :], 0.0)
    h = jnp.dot(h.astype(bf), w2_ref[...], preferred_element_type=jnp.float32)
    h = jnp.maximum(h + b2_ref[...][None, :], 0.0)
    r = jnp.dot(h.astype(bf), w3_ref[...], preferred_element_type=jnp.float32)
    out_ref[...] = r[:, 0] + b3_ref[0, 0]


def _mlp(nb, x, W0, b0, W1, b1, W2, b2, w3, b3):
    return pl.pallas_call(
        _mlp_body,
        grid=(nb // CHUNK,),
        in_specs=[
            pl.BlockSpec((CHUNK, D2), lambda i: (i, 0)),
            pl.BlockSpec((D2, 512), lambda i: (0, 0)),
            pl.BlockSpec((512,), lambda i: (0,)),
            pl.BlockSpec((512, 256), lambda i: (0, 0)),
            pl.BlockSpec((256,), lambda i: (0,)),
            pl.BlockSpec((256, 128), lambda i: (0, 0)),
            pl.BlockSpec((128,), lambda i: (0,)),
            pl.BlockSpec((128, 1), lambda i: (0, 0)),
            pl.BlockSpec((1, 1), lambda i: (0, 0)),
        ],
        out_specs=pl.BlockSpec((CHUNK,), lambda i: (i,)),
        out_shape=jax.ShapeDtypeStruct((nb,), jnp.float32),
        compiler_params=pltpu.CompilerParams(
            dimension_semantics=("parallel",),
        ),
    )(x, W0, b0, W1, b1, W2, b2, w3, b3)


NSPLIT = 2


def kernel(user_ids, book_ids, user_table, book_table,
           W0, b0, W1, b1, W2, b2, W3, b3):
    bf = jnp.bfloat16
    w = (W0.astype(bf), b0, W1.astype(bf), b1, W2.astype(bf), b2,
         W3.astype(bf), b3[None, :])
    h = B // NSPLIT
    uids2 = user_ids.astype(jnp.int32).reshape(-1, CH)
    bids2 = book_ids.astype(jnp.int32).reshape(-1, CH)
    rph = h // CH
    outs = []
    for s in range(NSPLIT):
        x = _make_gather(h, s * rph)(uids2, bids2, user_table, book_table)
        outs.append(_mlp(h, x, *w))
    return jnp.concatenate(outs, axis=0)


# speedup vs baseline: 3.0201x; 1.1054x over previous
import functools

import jax
import jax.numpy as jnp
from jax import lax
from jax.experimental import pallas as pl
from jax.experimental.pallas import tpu as pltpu
from jax.experimental.pallas import tpu_sc as plsc

B = 16384
D = 128
D2 = 2 * D
NC = 2
NS = 16
NW = NC * NS
CH = 128


@functools.cache
def _make_gather(nb, cbase0):
    bpw = nb // NW
    nch = bpw // CH
    mesh = plsc.VectorSubcoreMesh(core_axis_name="c", subcore_axis_name="s")

    @functools.partial(
        pl.kernel,
        mesh=mesh,
        out_type=jax.ShapeDtypeStruct((nb, D2), jnp.float32),
        scratch_types=[
            pltpu.VMEM((nch, CH), jnp.int32),
            pltpu.VMEM((nch, CH), jnp.int32),
            pltpu.VMEM((CH, D), jnp.float32),
            pltpu.VMEM((CH, D), jnp.float32),
            pltpu.VMEM((CH, D), jnp.float32),
            pltpu.VMEM((CH, D), jnp.float32),
            pltpu.SemaphoreType.DMA,
            pltpu.SemaphoreType.DMA,
        ],
    )
    def gather_k(uids2, bids2, utab, btab, x_out,
                 uidx_v, bidx_v, ur0, ur1, br0, br1, sg, sw):
        wid = lax.axis_index("s") * NC + lax.axis_index("c")
        cbase = wid * nch
        pltpu.sync_copy(uids2.at[pl.ds(cbase0 + cbase, nch)], uidx_v)
        pltpu.sync_copy(bids2.at[pl.ds(cbase0 + cbase, nch)], bidx_v)
        ubufs, bbufs = (ur0, ur1), (br0, br1)
        gathers = {}
        writes = {}

        def fire_gather(c):
            gu = pltpu.async_copy(utab.at[uidx_v.at[c]], ubufs[c % 2], sg)
            gb = pltpu.async_copy(btab.at[bidx_v.at[c]], bbufs[c % 2], sg)
            gathers[c] = (gu, gb)

        fire_gather(0)
        for c in range(nch):
            if c + 1 < nch:
                if c - 1 >= 0:
                    for w in writes.pop(c - 1):
                        w.wait()
                fire_gather(c + 1)
            gu, gb = gathers.pop(c)
            gu.wait()
            gb.wait()
            off = (cbase + c) * CH
            wu = pltpu.async_copy(
                ubufs[c % 2], x_out.at[pl.ds(off, CH), pl.ds(0, D)], sw)
            wb = pltpu.async_copy(
                bbufs[c % 2], x_out.at[pl.ds(off, CH), pl.ds(D, D)], sw)
            writes[c] = (wu, wb)
        for c in sorted(writes):
            for w in writes[c]:
                w.wait()

    return gather_k


CHUNK = 1024


def _mlp_body(x_ref, w0_ref, b0_ref, w1_ref, b1_ref,
              w2_ref, b2_ref, w3_ref, b3_ref, out_ref):
    bf = jnp.bfloat16
    h = jnp.dot(x_ref[...].astype(bf), w0_ref[...],
                preferred_element_type=jnp.float32)
    h = jnp.maximum(h + b0_ref[...][None, :], 0.0)
    h = jnp.dot(h.astype(bf), w1_ref[...], preferred_element_type=jnp.float32)
    h = jnp.maximum(h + b1_ref[...][None, :], 0.0)
    h = jnp.dot(h.astype(bf), w2_ref[...], preferred_element_type=jnp.float32)
    h = jnp.maximum(h + b2_ref[...][None, :], 0.0)
    r = jax.lax.dot_general(w3_ref[...], h.astype(bf),
                            dimension_numbers=(((1,), (1,)), ((), ())),
                            preferred_element_type=jnp.float32)
    out_ref[...] = r[0] + b3_ref[0, 0]


def _mlp(nb, x, W0, b0, W1, b1, W2, b2, w3, b3):
    return pl.pallas_call(
        _mlp_body,
        grid=(nb // CHUNK,),
        in_specs=[
            pl.BlockSpec((CHUNK, D2), lambda i: (i, 0)),
            pl.BlockSpec((D2, 512), lambda i: (0, 0)),
            pl.BlockSpec((512,), lambda i: (0,)),
            pl.BlockSpec((512, 256), lambda i: (0, 0)),
            pl.BlockSpec((256,), lambda i: (0,)),
            pl.BlockSpec((256, 128), lambda i: (0, 0)),
            pl.BlockSpec((128,), lambda i: (0,)),
            pl.BlockSpec((1, 128), lambda i: (0, 0)),
            pl.BlockSpec((1, 1), lambda i: (0, 0)),
        ],
        out_specs=pl.BlockSpec((CHUNK,), lambda i: (i,)),
        out_shape=jax.ShapeDtypeStruct((nb,), jnp.float32),
        compiler_params=pltpu.CompilerParams(
            dimension_semantics=("parallel",),
        ),
    )(x, W0, b0, W1, b1, W2, b2, w3, b3)


NSPLIT = 2


def kernel(user_ids, book_ids, user_table, book_table,
           W0, b0, W1, b1, W2, b2, W3, b3):
    bf = jnp.bfloat16
    w = (W0.astype(bf), b0, W1.astype(bf), b1, W2.astype(bf), b2,
         W3.reshape(1, 128).astype(bf), b3[None, :])
    h = B // NSPLIT
    uids2 = user_ids.astype(jnp.int32).reshape(-1, CH)
    bids2 = book_ids.astype(jnp.int32).reshape(-1, CH)
    rph = h // CH
    outs = []
    for s in range(NSPLIT):
        x = _make_gather(h, s * rph)(uids2, bids2, user_table, book_table)
        outs.append(_mlp(h, x, *w))
    return jnp.concatenate(outs, axis=0)
